# baseline (speedup 1.0000x reference)
import functools

import jax
import jax.numpy as jnp
from jax import lax
from jax.experimental import pallas as pl
from jax.experimental.pallas import tpu as pltpu
from jax.experimental.pallas import tpu_sc as plsc

_N = 100000
_NUM_SP = 1024
_CHUNK = 3136
_LAST_BASE = _N - _CHUNK
_NUM_CORES = 2


def _gather_body(tab, idx, out_t, out_r, idx_v, rows_v, sem):
    wid = lax.axis_index("s") * _NUM_CORES + lax.axis_index("c")
    base = jnp.minimum(wid * _CHUNK, _LAST_BASE)

    pltpu.sync_copy(idx.at[pl.ds(base, _CHUNK)], idx_v)
    pltpu.async_copy(tab.at[idx_v], rows_v, sem).wait()
    pltpu.sync_copy(rows_v.at[:, 0:3], out_t.at[pl.ds(base, _CHUNK)])
    pltpu.sync_copy(rows_v.at[:, 3:6], out_r.at[pl.ds(base, _CHUNK)])


def kernel(sp_delta_t, sp_delta_r, p2sp):
    tab = jnp.concatenate(
        [sp_delta_t, sp_delta_r, jnp.zeros((_NUM_SP, 2), jnp.float32)], axis=1)
    mesh = plsc.VectorSubcoreMesh(core_axis_name="c", subcore_axis_name="s")
    run = pl.kernel(
        _gather_body,
        mesh=mesh,
        compiler_params=pltpu.CompilerParams(use_tc_tiling_on_sc=False),
        out_type=(
            jax.ShapeDtypeStruct((_N, 3), jnp.float32),
            jax.ShapeDtypeStruct((_N, 3), jnp.float32),
        ),
        scratch_types=[
            pltpu.VMEM((_CHUNK,), jnp.int32),
            pltpu.VMEM((_CHUNK, 8), jnp.float32),
            pltpu.SemaphoreType.DMA,
        ],
    )
    return run(tab, p2sp)


# speedup vs baseline: 2.7495x; 2.7495x over previous
import functools

import jax
import jax.numpy as jnp
from jax import lax
from jax.experimental import pallas as pl
from jax.experimental.pallas import tpu as pltpu
from jax.experimental.pallas import tpu_sc as plsc

_N = 100000
_NUM_SP = 1024
_CHUNK = 3136
_LAST_BASE = _N - _CHUNK
_NUM_CORES = 2


def _gather_body(tab_t, tab_r, idx, out_t, out_r,
                 idx_v, rows_t_v, rows_r_v, sem_t, sem_r):
    wid = lax.axis_index("s") * _NUM_CORES + lax.axis_index("c")
    base = jnp.minimum(wid * _CHUNK, _LAST_BASE)

    pltpu.sync_copy(idx.at[pl.ds(base, _CHUNK)], idx_v)

    copy_t = pltpu.async_copy(tab_t.at[idx_v], rows_t_v, sem_t)
    copy_r = pltpu.async_copy(tab_r.at[idx_v], rows_r_v, sem_r)
    copy_t.wait()
    copy_r.wait()

    pltpu.sync_copy(rows_t_v, out_t.at[pl.ds(base, _CHUNK)])
    pltpu.sync_copy(rows_r_v, out_r.at[pl.ds(base, _CHUNK)])


def kernel(sp_delta_t, sp_delta_r, p2sp):
    mesh = plsc.VectorSubcoreMesh(core_axis_name="c", subcore_axis_name="s")
    run = pl.kernel(
        _gather_body,
        mesh=mesh,
        compiler_params=pltpu.CompilerParams(use_tc_tiling_on_sc=False),
        out_type=(
            jax.ShapeDtypeStruct((_N, 3), jnp.float32),
            jax.ShapeDtypeStruct((_N, 3), jnp.float32),
        ),
        scratch_types=[
            pltpu.VMEM((_CHUNK,), jnp.int32),
            pltpu.VMEM((_CHUNK, 3), jnp.float32),
            pltpu.VMEM((_CHUNK, 3), jnp.float32),
            pltpu.SemaphoreType.DMA,
            pltpu.SemaphoreType.DMA,
        ],
    )
    return run(sp_delta_t, sp_delta_r, p2sp)


# speedup vs baseline: 2.8546x; 1.0382x over previous
import functools

import jax
import jax.numpy as jnp
from jax import lax
from jax.experimental import pallas as pl
from jax.experimental.pallas import tpu as pltpu
from jax.experimental.pallas import tpu_sc as plsc

_N = 100000
_NUM_SP = 1024
_CHUNK = 3136
_LAST_BASE = _N - _CHUNK
_NUM_CORES = 2


def _gather_body(tab_t, tab_r, idx, out_t, out_r,
                 idx_v, rows_t_v, rows_r_v, sem_t, sem_r):
    wid = lax.axis_index("s") * _NUM_CORES + lax.axis_index("c")
    base = jnp.minimum(wid * _CHUNK, _LAST_BASE)

    pltpu.sync_copy(idx.at[pl.ds(base, _CHUNK)], idx_v)

    copy_t = pltpu.async_copy(tab_t.at[idx_v], rows_t_v, sem_t)
    copy_r = pltpu.async_copy(tab_r.at[idx_v], rows_r_v, sem_r)
    copy_t.wait()
    copy_r.wait()

    pltpu.sync_copy(rows_t_v, out_t.at[pl.ds(base, _CHUNK)])
    pltpu.sync_copy(rows_r_v, out_r.at[pl.ds(base, _CHUNK)])


def kernel(sp_delta_t, sp_delta_r, p2sp):
    pad = jnp.zeros((_NUM_SP, 1), jnp.float32)
    tab_t = jnp.concatenate([sp_delta_t, pad], axis=1)
    tab_r = jnp.concatenate([sp_delta_r, pad], axis=1)
    mesh = plsc.VectorSubcoreMesh(core_axis_name="c", subcore_axis_name="s")
    run = pl.kernel(
        _gather_body,
        mesh=mesh,
        compiler_params=pltpu.CompilerParams(use_tc_tiling_on_sc=False),
        out_type=(
            jax.ShapeDtypeStruct((_N, 4), jnp.float32),
            jax.ShapeDtypeStruct((_N, 4), jnp.float32),
        ),
        scratch_types=[
            pltpu.VMEM((_CHUNK,), jnp.int32),
            pltpu.VMEM((_CHUNK, 4), jnp.float32),
            pltpu.VMEM((_CHUNK, 4), jnp.float32),
            pltpu.SemaphoreType.DMA,
            pltpu.SemaphoreType.DMA,
        ],
    )
    out_t, out_r = run(tab_t, tab_r, p2sp)
    return out_t[:, :3], out_r[:, :3]
